# baseline (speedup 1.0000x reference)
import functools
import math

import jax
import jax.numpy as jnp
from jax import lax
from jax.experimental import pallas as pl
from jax.experimental.pallas import tpu as pltpu
from jax.experimental.pallas import tpu_sc as plsc

D_MODEL = 64
SCALE = math.sqrt(D_MODEL)

NC = 2
NS = 16
NW = NC * NS

B_TOT = 4096 * 50
B_PER_W = B_TOT // NW
CHUNK = 320
NCHUNK = B_PER_W // CHUNK

_mesh = plsc.VectorSubcoreMesh(core_axis_name="c", subcore_axis_name="s")


@functools.partial(
    pl.kernel,
    mesh=_mesh,
    out_type=jax.ShapeDtypeStruct((B_TOT // 2, 128), jnp.float32),
    scratch_types=[
        pltpu.VMEM((CHUNK,), jnp.int32),
        pltpu.VMEM((CHUNK,), jnp.int32),
        pltpu.VMEM((2, CHUNK, 128), jnp.float32),
        pltpu.VMEM((2, CHUNK // 2, 128), jnp.float32),
        pltpu.SemaphoreType.DMA,
        pltpu.SemaphoreType.DMA,
        pltpu.SemaphoreType.DMA,
        pltpu.SemaphoreType.DMA,
    ],
    compiler_params=pltpu.CompilerParams(needs_layout_passes=False),
)
def _embed(x_hbm, table_hbm, out_hbm, idx0_v, idx1_v, gath_v, out_v,
           isem0, isem1, gsem, osem):
    wid = lax.axis_index("s") * NC + lax.axis_index("c")
    base = wid * B_PER_W
    idxs = (idx0_v, idx1_v)
    isems = (isem0, isem1)

    def start_fetch(j, b):
        off = pl.multiple_of(base + j * CHUNK, CHUNK)
        pltpu.async_copy(x_hbm.at[pl.ds(off, CHUNK)], idxs[b], isems[b]
                         ).wait()
        pltpu.async_copy(table_hbm.at[idxs[b]], gath_v.at[b], gsem)

    def drain_gather(b):
        pltpu.make_async_copy(table_hbm.at[idxs[b]], gath_v.at[b], gsem
                              ).wait()

    def compact(j, b):
        @plsc.parallel_loop(0, CHUNK // 2, unroll=8)
        def pair(r2):
            for sub in range(2):
                for q in range(D_MODEL // 16):
                    out_v[b, r2, pl.ds(sub * D_MODEL + q * 16, 16)] = (
                        gath_v[b, 2 * r2 + sub, pl.ds(q * 16, 16)] * SCALE
                    )

    def store_out(j, b):
        off2 = pl.multiple_of((base + j * CHUNK) // 2, CHUNK // 2)
        pltpu.async_copy(out_v.at[b], out_hbm.at[pl.ds(off2, CHUNK // 2)],
                         osem)

    def drain_out(j, b):
        off2 = pl.multiple_of((base + j * CHUNK) // 2, CHUNK // 2)
        pltpu.make_async_copy(out_v.at[b], out_hbm.at[pl.ds(off2, CHUNK // 2)],
                              osem).wait()

    start_fetch(0, 0)

    def loop(j2, carry):
        j = j2 * 2
        start_fetch(j + 1, 1)
        drain_gather(0)
        compact(j, 0)
        lax.cond(j2 > 0, lambda: drain_out(j - 1, 1), lambda: None)
        store_out(j, 0)
        lax.cond(j2 + 1 < NCHUNK // 2,
                 lambda: start_fetch(j + 2, 0), lambda: None)
        drain_gather(1)
        compact(j + 1, 1)
        drain_out(j, 0)
        store_out(j + 1, 1)
        return carry

    lax.fori_loop(0, NCHUNK // 2, loop, 0)
    drain_out(NCHUNK - 1, 1)


def kernel(x, table):
    padded = jnp.pad(table, ((0, 0), (0, 128 - D_MODEL)))
    out = _embed(x.reshape(-1), padded)
    return out.reshape(x.shape + (D_MODEL,))


# speedup vs baseline: 1.0012x; 1.0012x over previous
import functools
import math

import jax
import jax.numpy as jnp
from jax import lax
from jax.experimental import pallas as pl
from jax.experimental.pallas import tpu as pltpu
from jax.experimental.pallas import tpu_sc as plsc

D_MODEL = 64
SCALE = math.sqrt(D_MODEL)

NC = 2
NS = 16
NW = NC * NS

B_TOT = 4096 * 50
B_PER_W = B_TOT // NW
CHUNK = 320
NCHUNK = B_PER_W // CHUNK

_mesh = plsc.VectorSubcoreMesh(core_axis_name="c", subcore_axis_name="s")


@functools.partial(
    pl.kernel,
    mesh=_mesh,
    out_type=jax.ShapeDtypeStruct((B_TOT // 2, 128), jnp.float32),
    scratch_types=[
        pltpu.VMEM((CHUNK,), jnp.int32),
        pltpu.VMEM((CHUNK,), jnp.int32),
        pltpu.VMEM((2, CHUNK, 128), jnp.float32),
        pltpu.VMEM((2, CHUNK // 2, 128), jnp.float32),
        pltpu.SemaphoreType.DMA,
        pltpu.SemaphoreType.DMA,
        pltpu.SemaphoreType.DMA,
        pltpu.SemaphoreType.DMA,
    ],
)
def _embed(x_hbm, table_hbm, out_hbm, idx0_v, idx1_v, gath_v, out_v,
           isem0, isem1, gsem, osem):
    wid = lax.axis_index("s") * NC + lax.axis_index("c")
    base = wid * B_PER_W
    idxs = (idx0_v, idx1_v)
    isems = (isem0, isem1)

    def start_fetch(j, b):
        off = pl.multiple_of(base + j * CHUNK, CHUNK)
        pltpu.async_copy(x_hbm.at[pl.ds(off, CHUNK)], idxs[b], isems[b]
                         ).wait()
        pltpu.async_copy(table_hbm.at[idxs[b]], gath_v.at[b], gsem)

    def drain_gather(b):
        pltpu.make_async_copy(table_hbm.at[idxs[b]], gath_v.at[b], gsem
                              ).wait()

    def compact(j, b):
        @plsc.parallel_loop(0, CHUNK // 2, unroll=8)
        def pair(r2):
            for sub in range(2):
                for q in range(D_MODEL // 16):
                    out_v[b, r2, pl.ds(sub * D_MODEL + q * 16, 16)] = (
                        gath_v[b, 2 * r2 + sub, pl.ds(q * 16, 16)] * SCALE
                    )

    def store_out(j, b):
        off2 = pl.multiple_of((base + j * CHUNK) // 2, CHUNK // 2)
        pltpu.async_copy(out_v.at[b], out_hbm.at[pl.ds(off2, CHUNK // 2)],
                         osem)

    def drain_out(j, b):
        off2 = pl.multiple_of((base + j * CHUNK) // 2, CHUNK // 2)
        pltpu.make_async_copy(out_v.at[b], out_hbm.at[pl.ds(off2, CHUNK // 2)],
                              osem).wait()

    start_fetch(0, 0)

    def loop(j2, carry):
        j = j2 * 2
        start_fetch(j + 1, 1)
        drain_gather(0)
        compact(j, 0)
        lax.cond(j2 > 0, lambda: drain_out(j - 1, 1), lambda: None)
        store_out(j, 0)
        lax.cond(j2 + 1 < NCHUNK // 2,
                 lambda: start_fetch(j + 2, 0), lambda: None)
        drain_gather(1)
        compact(j + 1, 1)
        drain_out(j, 0)
        store_out(j + 1, 1)
        return carry

    lax.fori_loop(0, NCHUNK // 2, loop, 0)
    drain_out(NCHUNK - 1, 1)


def kernel(x, table):
    padded = jnp.pad(table, ((0, 0), (0, 128 - D_MODEL)))
    out = _embed(x.reshape(-1), padded)
    return out.reshape(x.shape + (D_MODEL,))


# speedup vs baseline: 1.1980x; 1.1965x over previous
import functools
import math

import jax
import jax.numpy as jnp
from jax import lax
from jax.experimental import pallas as pl
from jax.experimental.pallas import tpu as pltpu
from jax.experimental.pallas import tpu_sc as plsc

D_MODEL = 64
SCALE = math.sqrt(D_MODEL)

NC = 2
NS = 16
NW = NC * NS

B0 = 4096
SEQ = 50
B_TOT = B0 * SEQ
B_PER_W = B_TOT // NW
I0_PER_W = B0 // NW

_mesh = plsc.VectorSubcoreMesh(core_axis_name="c", subcore_axis_name="s")


@functools.partial(
    pl.kernel,
    mesh=_mesh,
    out_type=jax.ShapeDtypeStruct((SEQ, D_MODEL, B0), jnp.float32),
    scratch_types=[
        pltpu.VMEM((B_PER_W,), jnp.int32),
        pltpu.VMEM((I0_PER_W,), jnp.int32),
        pltpu.VMEM((I0_PER_W,), jnp.int32),
        pltpu.VMEM((2, I0_PER_W, 128), jnp.float32),
        pltpu.VMEM((2, D_MODEL, 130), jnp.float32),
        pltpu.SemaphoreType.DMA,
        pltpu.SemaphoreType.DMA,
        pltpu.SemaphoreType.DMA,
    ],
    compiler_params=pltpu.CompilerParams(needs_layout_passes=False),
)
def _embed(x_hbm, table_hbm, out_hbm, xloc_v, idx0_v, idx1_v, gath_v,
           trans_v, xsem, gsem, osem):
    wid = lax.axis_index("s") * NC + lax.axis_index("c")
    base = pl.multiple_of(wid * B_PER_W, B_PER_W)
    c0 = pl.multiple_of(wid * I0_PER_W, I0_PER_W)
    iota = lax.iota(jnp.int32, 16)
    idxs = (idx0_v, idx1_v)

    pltpu.async_copy(x_hbm.at[pl.ds(base, B_PER_W)], xloc_v, xsem).wait()

    def start_gather(i1, b):
        def grp(g, c):
            pos = (g * 16 + iota) * SEQ + i1
            idxs[b][pl.ds(g * 16, 16)] = plsc.load_gather(xloc_v, [pos])
            return c

        lax.fori_loop(0, I0_PER_W // 16, grp, 0)
        pltpu.async_copy(table_hbm.at[idxs[b]], gath_v.at[b], gsem)

    def drain_gather(b):
        pltpu.make_async_copy(table_hbm.at[idxs[b]], gath_v.at[b], gsem
                              ).wait()

    def transpose_block(b):
        def grp(g, c):
            rows = g * 16 + iota

            @plsc.parallel_loop(0, 16, unroll=2)
            def rot(s):
                dsub = (iota + s) & 15
                for q in range(D_MODEL // 16):
                    d = q * 16 + dsub
                    vals = plsc.load_gather(gath_v.at[b], [rows, d])
                    plsc.store_scatter(trans_v.at[b], [d, rows], vals * SCALE)

            return c

        lax.fori_loop(0, I0_PER_W // 16, grp, 0)

    def store_out(i1, b):
        pltpu.async_copy(
            trans_v.at[b, :, pl.ds(0, 128)],
            out_hbm.at[i1, :, pl.ds(c0, 128)],
            osem,
        )

    def drain_out(i1, b):
        pltpu.make_async_copy(
            trans_v.at[b, :, pl.ds(0, 128)],
            out_hbm.at[i1, :, pl.ds(c0, 128)],
            osem,
        ).wait()

    start_gather(0, 0)

    def loop(h, carry):
        i1 = h * 2
        start_gather(i1 + 1, 1)
        drain_gather(0)
        lax.cond(h > 0, lambda: drain_out(i1 - 2, 0), lambda: None)
        transpose_block(0)
        store_out(i1, 0)
        lax.cond(i1 + 2 < SEQ, lambda: start_gather(i1 + 2, 0),
                 lambda: None)
        drain_gather(1)
        lax.cond(h > 0, lambda: drain_out(i1 - 1, 1), lambda: None)
        transpose_block(1)
        store_out(i1 + 1, 1)
        return carry

    lax.fori_loop(0, SEQ // 2, loop, 0)
    drain_out(SEQ - 2, 0)
    drain_out(SEQ - 1, 1)


def kernel(x, table):
    padded = jnp.pad(table, ((0, 0), (0, 128 - D_MODEL)))
    out = _embed(x.reshape(-1), padded)
    return out.transpose(2, 0, 1).reshape(x.shape + (D_MODEL,))
